# Initial kernel scaffold; baseline (speedup 1.0000x reference)
import jax
import jax.numpy as jnp
from jax.experimental import pallas as pl


def kernel(outputs, table):
    raise NotImplementedError("write your pallas kernel here")


# speedup vs baseline: 7.6723x; 7.6723x over previous
import numpy as np
import jax
import jax.numpy as jnp
from jax.experimental import pallas as pl
from jax.experimental.pallas import tpu as pltpu

BATCH = 16
HIDDEN = 768
NUM_NODES = 26
NODES_PAD = 128

_PATHS = [
    ([0, 2, 3], [-1, 1, 1]),
    ([0, 2, 3, 5, 6], [-1, 1, -1, 1, 1]),
    ([0, 2, 14, 15, 17, 18], [-1, -1, 1, -1, 1, 1]),
    ([0, 2, 14, 20, 21], [-1, -1, -1, 1, 1]),
    ([0, 2, 3, 5, 8, 9], [-1, 1, -1, -1, 1, 1]),
    ([0, 2, 14, 15], [-1, -1, 1, 1]),
    ([0, 2, 14, 20, 23, 24], [-1, -1, -1, -1, 1, 1]),
    ([0, 2, 3, 5, 8, 11, 12], [-1, 1, -1, -1, -1, 1, 1]),
]

_S = np.zeros((BATCH, NODES_PAD), dtype=np.float32)
_W = np.zeros((BATCH, NODES_PAD), dtype=np.float32)
for _b in range(BATCH):
    _idxs, _signs = _PATHS[_b % len(_PATHS)]
    for _i, _s in zip(_idxs, _signs):
        _S[_b, _i] = float(_s)
        _W[_b, _i] = -1.0 / (BATCH * len(_idxs))


def _body(h_ref, t_ref, w_ref, s_ref, o_ref):
    h = h_ref[...]
    t = t_ref[...]
    scores = jax.lax.dot_general(
        h, t, (((1,), (1,)), ((), ())),
        preferred_element_type=jnp.float32)
    loss = w_ref[...] * jnp.log(jax.nn.sigmoid(s_ref[...] * scores) + 1e-7)
    o_ref[...] = jnp.sum(loss, keepdims=True)


@jax.jit
def kernel(outputs, table):
    hidden = outputs[:, 0, :]
    table_pad = jnp.pad(table, ((0, NODES_PAD - NUM_NODES), (0, 0)))
    w = jnp.asarray(_W)
    s = jnp.asarray(_S)
    out = pl.pallas_call(
        _body,
        out_shape=jax.ShapeDtypeStruct((1, 1), jnp.float32),
    )(hidden, table_pad, w, s)
    return out[0, 0]


# speedup vs baseline: 8.1045x; 1.0563x over previous
import numpy as np
import jax
import jax.numpy as jnp
from jax.experimental import pallas as pl
from jax.experimental.pallas import tpu as pltpu

BATCH = 16
HIDDEN = 768
NUM_NODES = 26
NODES_PAD = 32

_PATHS = [
    ([0, 2, 3], [-1, 1, 1]),
    ([0, 2, 3, 5, 6], [-1, 1, -1, 1, 1]),
    ([0, 2, 14, 15, 17, 18], [-1, -1, 1, -1, 1, 1]),
    ([0, 2, 14, 20, 21], [-1, -1, -1, 1, 1]),
    ([0, 2, 3, 5, 8, 9], [-1, 1, -1, -1, 1, 1]),
    ([0, 2, 14, 15], [-1, -1, 1, 1]),
    ([0, 2, 14, 20, 23, 24], [-1, -1, -1, -1, 1, 1]),
    ([0, 2, 3, 5, 8, 11, 12], [-1, 1, -1, -1, -1, 1, 1]),
]

_S = np.zeros((BATCH, NODES_PAD), dtype=np.float32)
_W = np.zeros((BATCH, NODES_PAD), dtype=np.float32)
for _b in range(BATCH):
    _idxs, _signs = _PATHS[_b % len(_PATHS)]
    for _i, _s in zip(_idxs, _signs):
        _S[_b, _i] = float(_s)
        _W[_b, _i] = -1.0 / (BATCH * len(_idxs))


def _body(h_ref, t_ref, w_ref, s_ref, o_ref):
    h = h_ref[...]
    t = t_ref[...]
    scores = jax.lax.dot_general(
        h, t, (((1,), (1,)), ((), ())),
        preferred_element_type=jnp.float32)
    loss = w_ref[...] * jnp.log(jax.nn.sigmoid(s_ref[...] * scores) + 1e-7)
    o_ref[...] = jnp.sum(loss, keepdims=True)


@jax.jit
def kernel(outputs, table):
    hidden = outputs[:, 0, :]
    table_pad = jnp.pad(table, ((0, NODES_PAD - NUM_NODES), (0, 0)))
    w = jnp.asarray(_W)
    s = jnp.asarray(_S)
    out = pl.pallas_call(
        _body,
        out_shape=jax.ShapeDtypeStruct((1, 1), jnp.float32),
    )(hidden, table_pad, w, s)
    return out[0, 0]


# speedup vs baseline: 11.8380x; 1.4607x over previous
import numpy as np
import jax
import jax.numpy as jnp
from jax.experimental import pallas as pl
from jax.experimental.pallas import tpu as pltpu

BATCH = 16
HIDDEN = 768
NUM_NODES = 26

_PATHS = [
    ([0, 2, 3], [-1, 1, 1]),
    ([0, 2, 3, 5, 6], [-1, 1, -1, 1, 1]),
    ([0, 2, 14, 15, 17, 18], [-1, -1, 1, -1, 1, 1]),
    ([0, 2, 14, 20, 21], [-1, -1, -1, 1, 1]),
    ([0, 2, 3, 5, 8, 9], [-1, 1, -1, -1, 1, 1]),
    ([0, 2, 14, 15], [-1, -1, 1, 1]),
    ([0, 2, 14, 20, 23, 24], [-1, -1, -1, -1, 1, 1]),
    ([0, 2, 3, 5, 8, 11, 12], [-1, 1, -1, -1, -1, 1, 1]),
]

_S = np.zeros((BATCH, NUM_NODES), dtype=np.float32)
_W = np.zeros((BATCH, NUM_NODES), dtype=np.float32)
for _b in range(BATCH):
    _idxs, _signs = _PATHS[_b % len(_PATHS)]
    for _i, _s in zip(_idxs, _signs):
        _S[_b, _i] = float(_s)
        _W[_b, _i] = -1.0 / (BATCH * len(_idxs))


def _body(h_ref, t_ref, w_ref, s_ref, o_ref):
    h = h_ref[...]
    t = t_ref[...]
    scores = jax.lax.dot_general(
        h, t, (((1,), (1,)), ((), ())),
        preferred_element_type=jnp.float32)
    loss = w_ref[...] * jnp.log(jax.nn.sigmoid(s_ref[...] * scores) + 1e-7)
    o_ref[...] = jnp.sum(loss, keepdims=True)


@jax.jit
def kernel(outputs, table):
    hidden = outputs[:, 0, :]
    out = pl.pallas_call(
        _body,
        out_shape=jax.ShapeDtypeStruct((1, 1), jnp.float32),
    )(hidden, table, jnp.asarray(_W), jnp.asarray(_S))
    return out[0, 0]


# speedup vs baseline: 15.7770x; 1.3327x over previous
import numpy as np
import jax
import jax.numpy as jnp
from jax.experimental import pallas as pl
from jax.experimental.pallas import tpu as pltpu

BATCH = 16
HIDDEN = 768
NUM_NODES = 26

_PATHS = [
    ([0, 2, 3], [-1, 1, 1]),
    ([0, 2, 3, 5, 6], [-1, 1, -1, 1, 1]),
    ([0, 2, 14, 15, 17, 18], [-1, -1, 1, -1, 1, 1]),
    ([0, 2, 14, 20, 21], [-1, -1, -1, 1, 1]),
    ([0, 2, 3, 5, 8, 9], [-1, 1, -1, -1, 1, 1]),
    ([0, 2, 14, 15], [-1, -1, 1, 1]),
    ([0, 2, 14, 20, 23, 24], [-1, -1, -1, -1, 1, 1]),
    ([0, 2, 3, 5, 8, 11, 12], [-1, 1, -1, -1, -1, 1, 1]),
]

_S = np.zeros((BATCH, NUM_NODES), dtype=np.float32)
_W = np.zeros((BATCH, NUM_NODES), dtype=np.float32)
for _b in range(BATCH):
    _idxs, _signs = _PATHS[_b % len(_PATHS)]
    for _i, _s in zip(_idxs, _signs):
        _S[_b, _i] = float(_s)
        _W[_b, _i] = -1.0 / (BATCH * len(_idxs))


def _body(out_hbm_ref, t_ref, w_ref, s_ref, o_ref, h_vmem, sem):
    copy = pltpu.make_async_copy(out_hbm_ref.at[:, 0, :], h_vmem, sem)
    copy.start()
    copy.wait()
    h = h_vmem[...]
    t = t_ref[...]
    scores = jax.lax.dot_general(
        h, t, (((1,), (1,)), ((), ())),
        preferred_element_type=jnp.float32)
    loss = w_ref[...] * jnp.log(jax.nn.sigmoid(s_ref[...] * scores) + 1e-7)
    o_ref[...] = jnp.sum(loss, keepdims=True)


@jax.jit
def kernel(outputs, table):
    out = pl.pallas_call(
        _body,
        out_shape=jax.ShapeDtypeStruct((1, 1), jnp.float32),
        in_specs=[
            pl.BlockSpec(memory_space=pltpu.MemorySpace.HBM),
            pl.BlockSpec((NUM_NODES, HIDDEN), lambda: (0, 0)),
            pl.BlockSpec((BATCH, NUM_NODES), lambda: (0, 0)),
            pl.BlockSpec((BATCH, NUM_NODES), lambda: (0, 0)),
        ],
        out_specs=pl.BlockSpec((1, 1), lambda: (0, 0)),
        scratch_shapes=[
            pltpu.VMEM((BATCH, HIDDEN), jnp.float32),
            pltpu.SemaphoreType.DMA,
        ],
    )(outputs, table, jnp.asarray(_W), jnp.asarray(_S))
    return out[0, 0]
